# baseline (speedup 1.0000x reference)
import functools

import jax
import jax.numpy as jnp
from jax import lax
from jax.experimental import pallas as pl
from jax.experimental.pallas import tpu as pltpu
from jax.experimental.pallas import tpu_sc as plsc

_NEG = -1000000000.0
_BIG = 1 << 24
_K = 20
_KPAD = 32
_CH_PER_STEP = 8
_H = 512
_W = 512
_NC = 2
_NS = 16
_CH_PER_TILE = 4
_L = 16


def _tc_stage1(x_ref, bm_ref, rowmax_ref, rowpack_ref):
    x = x_ref[0]
    ninf = float("-inf")
    ch, h, w = x.shape

    pad_c = jnp.full((ch, h, 1), ninf, jnp.float32)
    hmax = jnp.maximum(
        x,
        jnp.maximum(
            jnp.concatenate([x[:, :, 1:], pad_c], axis=2),
            jnp.concatenate([pad_c, x[:, :, : w - 1]], axis=2),
        ),
    )
    pad_r = jnp.full((ch, 1, w), ninf, jnp.float32)
    pooled = jnp.maximum(
        hmax,
        jnp.maximum(
            jnp.concatenate([hmax[:, 1:, :], pad_r], axis=1),
            jnp.concatenate([pad_r, hmax[:, : h - 1, :]], axis=1),
        ),
    )

    is_peak = (x == pooled) & (x > 0)
    masked = jnp.where(is_peak, x, _NEG)

    ciota3 = lax.broadcasted_iota(jnp.int32, (ch, h, w), 2)
    riota3 = lax.broadcasted_iota(jnp.int32, (ch, h, w), 1)
    bits = jnp.where(
        is_peak, jnp.left_shift(1, jnp.bitwise_and(riota3, 15)), 0
    )
    bm_ref[0] = jnp.sum(bits.reshape(ch, h // 16, 16, w), axis=2)
    rowmax = jnp.max(masked, axis=2)
    rowarg = jnp.min(
        jnp.where(masked == rowmax[:, :, None], ciota3, _BIG), axis=2
    )
    riota = lax.broadcasted_iota(jnp.int32, (ch, h), 1)
    rowmax_ref[0] = rowmax
    rowpack_ref[0] = riota * w + rowarg


def _sc_stage2(
    conf2_hbm,
    bm2_hbm,
    rowmax_hbm,
    rowpack_hbm,
    vals_hbm,
    xs_hbm,
    ys_hbm,
    rm0,
    rm1,
    rm2,
    rm3,
    rp0,
    rp1,
    rp2,
    rp3,
    rv0,
    rv1,
    rv2,
    rv3,
    bb0,
    bb1,
    bb2,
    bb3,
    mrow,
    ob,
    sem0,
    sem1,
    sem2,
    sem3,
):
    rms = (rm0, rm1, rm2, rm3)
    rps = (rp0, rp1, rp2, rp3)
    rvs = (rv0, rv1, rv2, rv3)
    bbs = (bb0, bb1, bb2, bb3)
    sems = (sem0, sem1, sem2, sem3)
    lane_i = lax.broadcasted_iota(jnp.int32, (_L,), 0)

    def _perm(vec, idx):
        return vec.at[idx].get(mode="promise_in_bounds")

    def _max_scalar(vec):
        for s in (1, 2, 4, 8):
            vec = jnp.maximum(vec, _perm(vec, lane_i ^ s))
        return vec[0]

    def _min_scalar_i32(vec):
        for s in (1, 2, 4, 8):
            vec = jnp.minimum(vec, _perm(vec, lane_i ^ s))
        return vec[0]
    h, w, L = _H, _W, _L
    nvec_h = h // L
    nvec_w = w // L
    lane = lax.broadcasted_iota(jnp.int32, (L,), 0)
    ninf_v = jnp.full((L,), float("-inf"), jnp.float32)
    big_v = jnp.full((L,), _BIG, jnp.int32)

    cid = lax.axis_index("c")
    sid = lax.axis_index("s")
    wid = sid * _NC + cid
    ch_per_tile = _CH_PER_TILE

    chans = [wid * ch_per_tile + j for j in range(ch_per_tile)]
    for j in range(ch_per_tile):
        pltpu.sync_copy(rowmax_hbm.at[chans[j]], rms[j])
        pltpu.sync_copy(rowpack_hbm.at[chans[j]], rps[j])
        pltpu.sync_copy(bm2_hbm.at[chans[j]], bbs[j])
    nw_row = w // L

    def iter_body(i, carry):
        carry = list(carry)

        found = []
        for j in range(ch_per_tile):
            rm, rp = rms[j], rps[j]

            def _mx(t, a, rm=rm):
                return jnp.maximum(a, rm[pl.ds(t * L, L)])

            acc = lax.fori_loop(0, nvec_h, _mx, ninf_v)
            g = _max_scalar(acc)

            def _fx(t, a, rm=rm, rp=rp, g=g):
                m = rm[pl.ds(t * L, L)] == g
                return jnp.minimum(a, jnp.where(m, rp[pl.ds(t * L, L)], _BIG))

            fac = lax.fori_loop(0, nvec_h, _fx, big_v)
            fidx = _min_scalar_i32(fac)
            rstar = lax.div(fidx, w)
            cstar = lax.rem(fidx, w)
            pltpu.make_async_copy(
                conf2_hbm.at[chans[j] * h + rstar], rvs[j], sems[j]
            ).start()
            found.append((g, rstar, cstar))

        for j in range(ch_per_tile):
            g, rstar, cstar = found[j]
            rm, rp, rowv, bb = rms[j], rps[j], rvs[j], bbs[j]
            pltpu.make_async_copy(
                conf2_hbm.at[chans[j] * h + rstar], rowv, sems[j]
            ).wait()

            def _nm(t, a, rowv=rowv, bb=bb, g=g, rstar=rstar, cstar=cstar):
                v = rowv[pl.ds(t * L, L)]
                wbase = lax.div(rstar, L) * w + t * L
                wgrp = bb[pl.ds(wbase, L)]
                bit = lax.bitwise_and(
                    lax.shift_right_logical(wgrp, lax.rem(rstar, L)), 1
                )
                mv = jnp.where(bit == 1, v, _NEG)
                mrow[pl.ds(t * L, L)] = mv
                cio = lane + t * L
                elig = (mv < g) | ((mv == g) & (cio > cstar))
                return jnp.maximum(a, jnp.where(elig, mv, float("-inf")))

            nacc = lax.fori_loop(0, nvec_w, _nm, ninf_v)
            newmax = _max_scalar(nacc)

            def _na(t, a, g=g, cstar=cstar, newmax=newmax):
                v = mrow[pl.ds(t * L, L)]
                cio = lane + t * L
                elig = (v < g) | ((v == g) & (cio > cstar))
                m2 = elig & (v == newmax)
                return jnp.minimum(a, jnp.where(m2, cio, _BIG))

            aacc = lax.fori_loop(0, nvec_w, _na, big_v)
            newarg = _min_scalar_i32(aacc)

            base = lax.mul(lax.div(rstar, L), L)
            off = lax.rem(rstar, L)
            sel = lane == off
            rm[pl.ds(base, L)] = jnp.where(sel, newmax, rm[pl.ds(base, L)])
            rp[pl.ds(base, L)] = jnp.where(
                sel, rstar * w + newarg, rp[pl.ds(base, L)]
            )

            gf = g
            cf = cstar.astype(jnp.float32)
            rf = rstar.astype(jnp.float32)
            c0 = 6 * j
            carry[c0 + 0] = jnp.where(lane == i, gf, carry[c0 + 0])
            carry[c0 + 1] = jnp.where(lane == i - L, gf, carry[c0 + 1])
            carry[c0 + 2] = jnp.where(lane == i, cf, carry[c0 + 2])
            carry[c0 + 3] = jnp.where(lane == i - L, cf, carry[c0 + 3])
            carry[c0 + 4] = jnp.where(lane == i, rf, carry[c0 + 4])
            carry[c0 + 5] = jnp.where(lane == i - L, rf, carry[c0 + 5])
        return tuple(carry)

    zf = jnp.zeros((L,), jnp.float32)
    res = lax.fori_loop(
        0, _K, iter_body, tuple(zf for _ in range(6 * ch_per_tile))
    )

    for j in range(ch_per_tile):
        v0, v1, x0, x1, y0, y1 = res[6 * j : 6 * j + 6]
        ob[pl.ds(0, L)] = v0
        ob[pl.ds(L, L)] = v1
        pltpu.sync_copy(ob, vals_hbm.at[chans[j]])
        ob[pl.ds(0, L)] = x0
        ob[pl.ds(L, L)] = x1
        pltpu.sync_copy(ob, xs_hbm.at[chans[j]])
        ob[pl.ds(0, L)] = y0
        ob[pl.ds(L, L)] = y1
        pltpu.sync_copy(ob, ys_hbm.at[chans[j]])


def kernel(confmaps, k):
    b, n, h, w = confmaps.shape
    bc = b * n
    steps = bc // _CH_PER_STEP
    xin = confmaps.reshape(steps, _CH_PER_STEP, h, w)

    bm, rowmax, rowpack = pl.pallas_call(
        _tc_stage1,
        grid=(steps,),
        in_specs=[pl.BlockSpec((1, _CH_PER_STEP, h, w), lambda i: (i, 0, 0, 0))],
        out_specs=[
            pl.BlockSpec((1, _CH_PER_STEP, h // _L, w), lambda i: (i, 0, 0, 0)),
            pl.BlockSpec((1, _CH_PER_STEP, h), lambda i: (i, 0, 0)),
            pl.BlockSpec((1, _CH_PER_STEP, h), lambda i: (i, 0, 0)),
        ],
        out_shape=[
            jax.ShapeDtypeStruct((steps, _CH_PER_STEP, h // _L, w), jnp.int32),
            jax.ShapeDtypeStruct((steps, _CH_PER_STEP, h), jnp.float32),
            jax.ShapeDtypeStruct((steps, _CH_PER_STEP, h), jnp.int32),
        ],
        compiler_params=pltpu.CompilerParams(
            dimension_semantics=("arbitrary",),
        ),
    )(xin)

    conf2 = confmaps.reshape(bc * h, w)
    bm2 = bm.reshape(bc, (h // _L) * w)
    rowmax2 = rowmax.reshape(bc, h)
    rowpack2 = rowpack.reshape(bc, h)

    mesh = plsc.VectorSubcoreMesh(core_axis_name="c", subcore_axis_name="s")
    out_t = jax.ShapeDtypeStruct((bc, _KPAD), jnp.float32)
    sc_fn = functools.partial(
        pl.kernel,
        mesh=mesh,
        out_type=[out_t, out_t, out_t],
        scratch_types=(
            [pltpu.VMEM((h,), jnp.float32) for _ in range(_CH_PER_TILE)]
            + [pltpu.VMEM((h,), jnp.int32) for _ in range(_CH_PER_TILE)]
            + [pltpu.VMEM((w,), jnp.float32) for _ in range(_CH_PER_TILE)]
            + [pltpu.VMEM(((h // _L) * w,), jnp.int32) for _ in range(_CH_PER_TILE)]
            + [pltpu.VMEM((w,), jnp.float32)]
            + [pltpu.VMEM((_KPAD,), jnp.float32)]
            + [pltpu.SemaphoreType.DMA for _ in range(_CH_PER_TILE)]
        ),
    )(_sc_stage2)
    vals, xs, ys = sc_fn(conf2, bm2, rowmax2, rowpack2)

    values = vals[:, :_K].reshape(b, n, _K)
    xcoord = xs[:, :_K].reshape(b, n, _K)
    ycoord = ys[:, :_K].reshape(b, n, _K)
    peaks = jnp.stack([xcoord, ycoord], axis=-1)
    valid = (values > 0) & (jnp.arange(_K) < k)
    return (peaks, values, valid)


# speedup vs baseline: 1.3101x; 1.3101x over previous
import functools

import jax
import jax.numpy as jnp
from jax import lax
from jax.experimental import pallas as pl
from jax.experimental.pallas import tpu as pltpu
from jax.experimental.pallas import tpu_sc as plsc

_NEG = -1000000000.0
_BIG = 1 << 24
_K = 20
_KPAD = 32
_CH_PER_STEP = 8
_H = 512
_W = 512
_NC = 2
_NS = 16
_CH_PER_TILE = 4
_L = 16


def _tc_stage1(x_ref, masked_ref, rowmax_ref, rowpack_ref):
    x = x_ref[0]
    ninf = float("-inf")
    ch, h, w = x.shape

    pad_c = jnp.full((ch, h, 1), ninf, jnp.float32)
    hmax = jnp.maximum(
        x,
        jnp.maximum(
            jnp.concatenate([x[:, :, 1:], pad_c], axis=2),
            jnp.concatenate([pad_c, x[:, :, : w - 1]], axis=2),
        ),
    )
    pad_r = jnp.full((ch, 1, w), ninf, jnp.float32)
    pooled = jnp.maximum(
        hmax,
        jnp.maximum(
            jnp.concatenate([hmax[:, 1:, :], pad_r], axis=1),
            jnp.concatenate([pad_r, hmax[:, : h - 1, :]], axis=1),
        ),
    )

    is_peak = (x == pooled) & (x > 0)
    masked = jnp.where(is_peak, x, _NEG)
    masked_ref[0] = masked

    ciota3 = lax.broadcasted_iota(jnp.int32, (ch, h, w), 2)
    rowmax = jnp.max(masked, axis=2)
    rowarg = jnp.min(
        jnp.where(masked == rowmax[:, :, None], ciota3, _BIG), axis=2
    )
    riota = lax.broadcasted_iota(jnp.int32, (ch, h), 1)
    rowmax_ref[0] = rowmax
    rowpack_ref[0] = riota * w + rowarg


def _sc_stage2(
    masked2_hbm,
    rowmax_hbm,
    rowpack_hbm,
    vals_hbm,
    xs_hbm,
    ys_hbm,
    rm0,
    rm1,
    rm2,
    rm3,
    rp0,
    rp1,
    rp2,
    rp3,
    rv0,
    rv1,
    rv2,
    rv3,
    ob,
    sem0,
    sem1,
    sem2,
    sem3,
):
    rms = (rm0, rm1, rm2, rm3)
    rps = (rp0, rp1, rp2, rp3)
    rvs = (rv0, rv1, rv2, rv3)
    sems = (sem0, sem1, sem2, sem3)
    lane_i = lax.broadcasted_iota(jnp.int32, (_L,), 0)

    def _perm(vec, idx):
        return vec.at[idx].get(mode="promise_in_bounds")

    def _max_scalar(vec):
        for s in (1, 2, 4, 8):
            vec = jnp.maximum(vec, _perm(vec, lane_i ^ s))
        return vec[0]

    def _min_scalar_i32(vec):
        for s in (1, 2, 4, 8):
            vec = jnp.minimum(vec, _perm(vec, lane_i ^ s))
        return vec[0]
    h, w, L = _H, _W, _L
    nvec_h = h // L
    nvec_w = w // L
    lane = lax.broadcasted_iota(jnp.int32, (L,), 0)
    ninf_v = jnp.full((L,), float("-inf"), jnp.float32)
    big_v = jnp.full((L,), _BIG, jnp.int32)

    cid = lax.axis_index("c")
    sid = lax.axis_index("s")
    wid = sid * _NC + cid
    ch_per_tile = _CH_PER_TILE

    chans = [wid * ch_per_tile + j for j in range(ch_per_tile)]
    for j in range(ch_per_tile):
        pltpu.sync_copy(rowmax_hbm.at[chans[j]], rms[j])
        pltpu.sync_copy(rowpack_hbm.at[chans[j]], rps[j])

    def iter_body(i, carry):
        carry = list(carry)

        found = []
        for j in range(ch_per_tile):
            rm, rp = rms[j], rps[j]

            def _mx(t, a, rm=rm):
                for k in range(4):
                    a = jnp.maximum(a, rm[pl.ds((t * 4 + k) * L, L)])
                return a

            acc = lax.fori_loop(0, nvec_h // 4, _mx, ninf_v)
            g = _max_scalar(acc)

            def _fx(t, a, rm=rm, rp=rp, g=g):
                for k in range(4):
                    tt = t * 4 + k
                    m = rm[pl.ds(tt * L, L)] == g
                    a = jnp.minimum(a, jnp.where(m, rp[pl.ds(tt * L, L)], _BIG))
                return a

            fac = lax.fori_loop(0, nvec_h // 4, _fx, big_v)
            fidx = _min_scalar_i32(fac)
            rstar = lax.div(fidx, w)
            cstar = lax.rem(fidx, w)
            pltpu.make_async_copy(
                masked2_hbm.at[chans[j] * h + rstar], rvs[j], sems[j]
            ).start()
            found.append((g, rstar, cstar))

        for j in range(ch_per_tile):
            g, rstar, cstar = found[j]
            rm, rp, rowv = rms[j], rps[j], rvs[j]
            pltpu.make_async_copy(
                masked2_hbm.at[chans[j] * h + rstar], rowv, sems[j]
            ).wait()

            def _nm(t, a, rowv=rowv, g=g, cstar=cstar):
                for k in range(4):
                    tt = t * 4 + k
                    v = rowv[pl.ds(tt * L, L)]
                    cio = lane + tt * L
                    elig = (v < g) | ((v == g) & (cio > cstar))
                    a = jnp.maximum(a, jnp.where(elig, v, float("-inf")))
                return a

            nacc = lax.fori_loop(0, nvec_w // 4, _nm, ninf_v)
            newmax = _max_scalar(nacc)

            def _na(t, a, rowv=rowv, g=g, cstar=cstar, newmax=newmax):
                for k in range(4):
                    tt = t * 4 + k
                    v = rowv[pl.ds(tt * L, L)]
                    cio = lane + tt * L
                    elig = (v < g) | ((v == g) & (cio > cstar))
                    m2 = elig & (v == newmax)
                    a = jnp.minimum(a, jnp.where(m2, cio, _BIG))
                return a

            aacc = lax.fori_loop(0, nvec_w // 4, _na, big_v)
            newarg = _min_scalar_i32(aacc)

            base = lax.mul(lax.div(rstar, L), L)
            off = lax.rem(rstar, L)
            sel = lane == off
            rm[pl.ds(base, L)] = jnp.where(sel, newmax, rm[pl.ds(base, L)])
            rp[pl.ds(base, L)] = jnp.where(
                sel, rstar * w + newarg, rp[pl.ds(base, L)]
            )

            gf = g
            cf = cstar.astype(jnp.float32)
            rf = rstar.astype(jnp.float32)
            c0 = 6 * j
            carry[c0 + 0] = jnp.where(lane == i, gf, carry[c0 + 0])
            carry[c0 + 1] = jnp.where(lane == i - L, gf, carry[c0 + 1])
            carry[c0 + 2] = jnp.where(lane == i, cf, carry[c0 + 2])
            carry[c0 + 3] = jnp.where(lane == i - L, cf, carry[c0 + 3])
            carry[c0 + 4] = jnp.where(lane == i, rf, carry[c0 + 4])
            carry[c0 + 5] = jnp.where(lane == i - L, rf, carry[c0 + 5])
        return tuple(carry)

    zf = jnp.zeros((L,), jnp.float32)
    res = lax.fori_loop(
        0, _K, iter_body, tuple(zf for _ in range(6 * ch_per_tile))
    )

    for j in range(ch_per_tile):
        v0, v1, x0, x1, y0, y1 = res[6 * j : 6 * j + 6]
        ob[pl.ds(0, L)] = v0
        ob[pl.ds(L, L)] = v1
        pltpu.sync_copy(ob, vals_hbm.at[chans[j]])
        ob[pl.ds(0, L)] = x0
        ob[pl.ds(L, L)] = x1
        pltpu.sync_copy(ob, xs_hbm.at[chans[j]])
        ob[pl.ds(0, L)] = y0
        ob[pl.ds(L, L)] = y1
        pltpu.sync_copy(ob, ys_hbm.at[chans[j]])


def kernel(confmaps, k):
    b, n, h, w = confmaps.shape
    bc = b * n
    steps = bc // _CH_PER_STEP
    xin = confmaps.reshape(steps, _CH_PER_STEP, h, w)

    masked, rowmax, rowpack = pl.pallas_call(
        _tc_stage1,
        grid=(steps,),
        in_specs=[pl.BlockSpec((1, _CH_PER_STEP, h, w), lambda i: (i, 0, 0, 0))],
        out_specs=[
            pl.BlockSpec((1, _CH_PER_STEP, h, w), lambda i: (i, 0, 0, 0)),
            pl.BlockSpec((1, _CH_PER_STEP, h), lambda i: (i, 0, 0)),
            pl.BlockSpec((1, _CH_PER_STEP, h), lambda i: (i, 0, 0)),
        ],
        out_shape=[
            jax.ShapeDtypeStruct((steps, _CH_PER_STEP, h, w), jnp.float32),
            jax.ShapeDtypeStruct((steps, _CH_PER_STEP, h), jnp.float32),
            jax.ShapeDtypeStruct((steps, _CH_PER_STEP, h), jnp.int32),
        ],
        compiler_params=pltpu.CompilerParams(
            dimension_semantics=("arbitrary",),
        ),
    )(xin)

    masked2 = masked.reshape(bc * h, w)
    rowmax2 = rowmax.reshape(bc, h)
    rowpack2 = rowpack.reshape(bc, h)

    mesh = plsc.VectorSubcoreMesh(core_axis_name="c", subcore_axis_name="s")
    out_t = jax.ShapeDtypeStruct((bc, _KPAD), jnp.float32)
    sc_fn = functools.partial(
        pl.kernel,
        mesh=mesh,
        out_type=[out_t, out_t, out_t],
        scratch_types=(
            [pltpu.VMEM((h,), jnp.float32) for _ in range(_CH_PER_TILE)]
            + [pltpu.VMEM((h,), jnp.int32) for _ in range(_CH_PER_TILE)]
            + [pltpu.VMEM((w,), jnp.float32) for _ in range(_CH_PER_TILE)]
            + [pltpu.VMEM((_KPAD,), jnp.float32)]
            + [pltpu.SemaphoreType.DMA for _ in range(_CH_PER_TILE)]
        ),
    )(_sc_stage2)
    vals, xs, ys = sc_fn(masked2, rowmax2, rowpack2)

    values = vals[:, :_K].reshape(b, n, _K)
    xcoord = xs[:, :_K].reshape(b, n, _K)
    ycoord = ys[:, :_K].reshape(b, n, _K)
    peaks = jnp.stack([xcoord, ycoord], axis=-1)
    valid = (values > 0) & (jnp.arange(_K) < k)
    return (peaks, values, valid)
